# baseline (speedup 1.0000x reference)
import functools

import jax
import jax.numpy as jnp
from jax.experimental import pallas as pl


def _body(cand_ref, scores_ref, mask_ref, partial_ref, final_ref, weights_ref):
    s = scores_ref[0]
    K = s.shape[0]
    best = jnp.argmax(s, axis=0)
    kidx = jax.lax.broadcasted_iota(jnp.int32, s.shape, 0)
    onehot = (kidx == best[None]).astype(jnp.float32)
    weights_ref[0] = onehot
    cand = cand_ref[0]
    sel = (cand * onehot[:, None]).sum(axis=0)
    m = mask_ref[0, 0]
    final_ref[0] = partial_ref[0] * m[None] + sel * (1.0 - m[None])


@functools.partial(jax.jit, static_argnames=("nh",))
def _run(candidate_images, selection_scores, mask, partial_image, nh=384):
    B, K, C, H, W = candidate_images.shape
    grid = (B, H // nh)
    out_shapes = (
        jax.ShapeDtypeStruct((B, C, H, W), jnp.float32),
        jax.ShapeDtypeStruct((B, K, H, W), jnp.float32),
    )
    return pl.pallas_call(
        _body,
        grid=grid,
        in_specs=[
            pl.BlockSpec((1, K, C, nh, W), lambda b, j: (b, 0, 0, j, 0)),
            pl.BlockSpec((1, K, nh, W), lambda b, j: (b, 0, j, 0)),
            pl.BlockSpec((1, 1, nh, W), lambda b, j: (b, 0, j, 0)),
            pl.BlockSpec((1, C, nh, W), lambda b, j: (b, 0, j, 0)),
        ],
        out_specs=(
            pl.BlockSpec((1, C, nh, W), lambda b, j: (b, 0, j, 0)),
            pl.BlockSpec((1, K, nh, W), lambda b, j: (b, 0, j, 0)),
        ),
        out_shape=out_shapes,
        compiler_params=__import__("jax.experimental.pallas.tpu", fromlist=["x"]).CompilerParams(dimension_semantics=("parallel", "arbitrary")),
    )(candidate_images, selection_scores, mask, partial_image)


def kernel(candidate_images, selection_scores, mask, partial_image):
    return _run(candidate_images, selection_scores, mask, partial_image)
